# baseline (speedup 1.0000x reference)
import jax
import jax.numpy as jnp
from jax.experimental import pallas as pl
from jax.experimental.pallas import tpu as pltpu

_V = 100001
_MASK_IDX = -1
_EPS = 1e-3
_B = 16
_L = 16
_R = _B * _L
_BZ = 16
_NB = _R // _BZ


def _rotl(x, d):
    return jax.lax.shift_left(x, d) | jax.lax.shift_right_logical(x, 32 - d)


def _threefry_bits(cnt):
    ks0 = jnp.int32(0)
    ks1 = jnp.int32(1)
    ks2 = jnp.int32(0x1BD11BDB)
    ks = (ks0, ks1, ks2)
    rots = ((13, 15, 26, 6), (17, 29, 16, 24))
    x0 = jnp.zeros_like(cnt) + ks0
    x1 = cnt + ks1
    for i in range(5):
        for r in rots[i % 2]:
            x0 = x0 + x1
            x1 = _rotl(x1, r)
            x1 = x0 ^ x1
        x0 = x0 + ks[(i + 1) % 3]
        x1 = x1 + ks[(i + 2) % 3] + jnp.int32(i + 1)
    return x0 ^ x1


def _body(blk_ref, xtm_ref, xto_ref, sig_ref, step_ref, x_hbm_ref,
          rev_hbm_ref, nxt_ref, zbuf_ref, xbuf_ref, revbuf_ref,
          zsem, wsem, rsem):
    nxt_ref[...] = xto_ref[...]
    zbuf_ref[...] = jnp.zeros((_BZ, _V), jnp.float32)

    for i in range(_NB):
        masked = blk_ref[i] != 0

        @pl.when(jnp.logical_not(masked))
        def _():
            pltpu.make_async_copy(
                zbuf_ref, rev_hbm_ref.at[pl.ds(i * _BZ, _BZ), :],
                zsem.at[i]).start()

        @pl.when(masked)
        def _():
            cp_in = pltpu.make_async_copy(
                x_hbm_ref.at[pl.ds(i * _BZ, _BZ), :], xbuf_ref, rsem)
            cp_in.start()
            cp_in.wait()
            x = xbuf_ref[...]
            score = jnp.exp(x)
            sig = sig_ref[pl.ds(i * _BZ, _BZ), :]
            xtm = xtm_ref[pl.ds(i * _BZ, _BZ), :]
            vv = jax.lax.broadcasted_iota(jnp.int32, (_BZ, _V), 1)
            is_last = vv == _V - 1
            masked_row = xtm == _V - 1
            s = jnp.sum(jnp.where(is_last, 0.0, score), axis=1, keepdims=True)
            rev = sig * jnp.where(is_last, -s, score)
            rev = jnp.where(masked_row, rev, 0.0)
            row = i * _BZ + jax.lax.broadcasted_iota(jnp.int32, (_BZ, _V), 0)
            bits = _threefry_bits(row * _V + vv)
            fbits = jax.lax.bitcast_convert_type(
                jax.lax.shift_right_logical(bits, 9) | jnp.int32(0x3F800000),
                jnp.float32) - 1.0
            u = jnp.maximum(fbits, 0.0)
            noise = 1e-6 - jnp.log(1e-6 + (1.0 - 1e-6) * u)
            step = step_ref[0]
            xt_prob = jnp.where(is_last, 1.0 + step * rev, step * rev)
            vals = xt_prob / noise
            m = jnp.max(vals, axis=1, keepdims=True)
            idx = jnp.min(jnp.where(vals == m, vv, _V), axis=1, keepdims=True)
            nxt_ref[pl.ds(i * _BZ, _BZ), :] = jnp.where(
                masked_row, idx, xto_ref[pl.ds(i * _BZ, _BZ), :])
            revbuf_ref[...] = rev
            cp_out = pltpu.make_async_copy(
                revbuf_ref, rev_hbm_ref.at[pl.ds(i * _BZ, _BZ), :], wsem)
            cp_out.start()
            cp_out.wait()

    for i in range(_NB):
        masked = blk_ref[i] != 0

        @pl.when(jnp.logical_not(masked))
        def _():
            pltpu.make_async_copy(
                zbuf_ref, rev_hbm_ref.at[pl.ds(i * _BZ, _BZ), :],
                zsem.at[i]).wait()


def kernel(output, xt, t, step_size):
    xt = xt.astype(jnp.int32)
    xtm = jnp.where(xt == _MASK_IDX, _V - 1, xt).reshape(_R, 1)
    blk = (jnp.max(xtm.reshape(_NB, _BZ), axis=1) == _V - 1).astype(jnp.int32)
    sigma = ((1.0 - _EPS) / (1.0 - (1.0 - _EPS) * t)).astype(jnp.float32)
    sigma_rows = jnp.repeat(sigma, _L).reshape(_R, 1)
    x2d = output.reshape(_R, _V)
    step = step_size.astype(jnp.float32)

    rev2d, nxt = pl.pallas_call(
        _body,
        in_specs=[
            pl.BlockSpec(memory_space=pltpu.SMEM),
            pl.BlockSpec((_R, 1), lambda: (0, 0)),
            pl.BlockSpec((_R, 1), lambda: (0, 0)),
            pl.BlockSpec((_R, 1), lambda: (0, 0)),
            pl.BlockSpec(memory_space=pltpu.SMEM),
            pl.BlockSpec(memory_space=pl.ANY),
        ],
        out_specs=[
            pl.BlockSpec(memory_space=pl.ANY),
            pl.BlockSpec((_R, 1), lambda: (0, 0)),
        ],
        out_shape=[
            jax.ShapeDtypeStruct((_R, _V), jnp.float32),
            jax.ShapeDtypeStruct((_R, 1), jnp.int32),
        ],
        scratch_shapes=[
            pltpu.VMEM((_BZ, _V), jnp.float32),
            pltpu.VMEM((_BZ, _V), jnp.float32),
            pltpu.VMEM((_BZ, _V), jnp.float32),
            pltpu.SemaphoreType.DMA((_NB,)),
            pltpu.SemaphoreType.DMA,
            pltpu.SemaphoreType.DMA,
        ],
    )(blk, xtm, xtm, sigma_rows, step, x2d)

    new_xt = jnp.where(nxt.reshape(_B, _L) == _V - 1, _MASK_IDX,
                       nxt.reshape(_B, _L))
    return new_xt, rev2d.reshape(_B, _L, _V)


# speedup vs baseline: 6.1272x; 6.1272x over previous
import jax
import jax.numpy as jnp
from jax.experimental import pallas as pl
from jax.experimental.pallas import tpu as pltpu

_V = 100001
_MASK_IDX = -1
_EPS = 1e-3
_B = 16
_L = 16
_R = _B * _L
_BR = 32
_G = _R // _BR


def _rotl(x, d):
    return jax.lax.shift_left(x, d) | jax.lax.shift_right_logical(x, 32 - d)


def _threefry_bits(cnt):
    ks0 = jnp.int32(0)
    ks1 = jnp.int32(1)
    ks2 = jnp.int32(0x1BD11BDB)
    ks = (ks0, ks1, ks2)
    rots = ((13, 15, 26, 6), (17, 29, 16, 24))
    x0 = jnp.zeros_like(cnt) + ks0
    x1 = cnt + ks1
    for i in range(5):
        for r in rots[i % 2]:
            x0 = x0 + x1
            x1 = _rotl(x1, r)
            x1 = x0 ^ x1
        x0 = x0 + ks[(i + 1) % 3]
        x1 = x1 + ks[(i + 2) % 3] + jnp.int32(i + 1)
    return x0 ^ x1


def _body(blk_ref, xtm_ref, sig_ref, step_ref, x_hbm_ref, rev_ref, nxt_ref,
          xbuf_ref, copy_sem):
    j = pl.program_id(0)
    xtm = xtm_ref[...]
    any_masked = blk_ref[j] != 0
    dirty = j < 4
    for back in range(1, 5):
        dirty = jnp.logical_or(
            dirty, blk_ref[jnp.maximum(j - back, 0)] != 0)

    @pl.when(jnp.logical_and(jnp.logical_not(any_masked), dirty))
    def _():
        rev_ref[...] = jnp.zeros((_BR, _V), jnp.float32)

    @pl.when(jnp.logical_not(any_masked))
    def _():
        nxt_ref[...] = xtm_ref[...]

    @pl.when(any_masked)
    def _():
        copy = pltpu.make_async_copy(
            x_hbm_ref.at[pl.ds(j * _BR, _BR), :], xbuf_ref, copy_sem)
        copy.start()
        copy.wait()
        x = xbuf_ref[...]
        score = jnp.exp(x)
        sig = sig_ref[...]
        vv = jax.lax.broadcasted_iota(jnp.int32, (_BR, _V), 1)
        is_last = vv == _V - 1
        masked_row = xtm == _V - 1
        s = jnp.sum(jnp.where(is_last, 0.0, score), axis=1, keepdims=True)
        rev = sig * jnp.where(is_last, -s, score)
        rev = jnp.where(masked_row, rev, 0.0)
        rev_ref[...] = rev
        row = j * _BR + jax.lax.broadcasted_iota(jnp.int32, (_BR, _V), 0)
        bits = _threefry_bits(row * _V + vv)
        fbits = jax.lax.bitcast_convert_type(
            jax.lax.shift_right_logical(bits, 9) | jnp.int32(0x3F800000),
            jnp.float32) - 1.0
        u = jnp.maximum(fbits, 0.0)
        noise = 1e-6 - jnp.log(1e-6 + (1.0 - 1e-6) * u)
        step = step_ref[0]
        xt_prob = jnp.where(is_last, 1.0 + step * rev, step * rev)
        vals = xt_prob / noise
        m = jnp.max(vals, axis=1, keepdims=True)
        idx = jnp.min(jnp.where(vals == m, vv, _V), axis=1, keepdims=True)
        nxt_ref[...] = jnp.where(masked_row, idx, xtm_ref[...])


def kernel(output, xt, t, step_size):
    xt = xt.astype(jnp.int32)
    xtm = jnp.where(xt == _MASK_IDX, _V - 1, xt).reshape(_R, 1)
    blk = (jnp.max(xtm.reshape(_G, _BR), axis=1) == _V - 1).astype(jnp.int32)
    sigma = ((1.0 - _EPS) / (1.0 - (1.0 - _EPS) * t)).astype(jnp.float32)
    sigma_rows = jnp.repeat(sigma, _L).reshape(_R, 1)
    x2d = output.reshape(_R, _V)
    step = step_size.astype(jnp.float32)

    rev2d, nxt = pl.pallas_call(
        _body,
        grid=(_G,),
        in_specs=[
            pl.BlockSpec(memory_space=pltpu.SMEM),
            pl.BlockSpec((_BR, 1), lambda j: (j, 0)),
            pl.BlockSpec((_BR, 1), lambda j: (j, 0)),
            pl.BlockSpec(memory_space=pltpu.SMEM),
            pl.BlockSpec(memory_space=pl.ANY),
        ],
        out_specs=[
            pl.BlockSpec((_BR, _V), lambda j: (j, 0)),
            pl.BlockSpec((_BR, 1), lambda j: (j, 0)),
        ],
        out_shape=[
            jax.ShapeDtypeStruct((_R, _V), jnp.float32),
            jax.ShapeDtypeStruct((_R, 1), jnp.int32),
        ],
        scratch_shapes=[
            pltpu.VMEM((_BR, _V), jnp.float32),
            pltpu.SemaphoreType.DMA,
        ],
    )(blk, xtm, sigma_rows, step, x2d)

    new_xt = jnp.where(nxt.reshape(_B, _L) == _V - 1, _MASK_IDX,
                       nxt.reshape(_B, _L))
    return new_xt, rev2d.reshape(_B, _L, _V)


# speedup vs baseline: 6.1517x; 1.0040x over previous
import jax
import jax.numpy as jnp
from jax.experimental import pallas as pl
from jax.experimental.pallas import tpu as pltpu

_V = 100001
_MASK_IDX = -1
_EPS = 1e-3
_B = 16
_L = 16
_R = _B * _L
_BR = 32
_BS = 16
_G = _R // _BR


def _rotl(x, d):
    return jax.lax.shift_left(x, d) | jax.lax.shift_right_logical(x, 32 - d)


def _threefry_bits(cnt):
    ks0 = jnp.int32(0)
    ks1 = jnp.int32(1)
    ks2 = jnp.int32(0x1BD11BDB)
    ks = (ks0, ks1, ks2)
    rots = ((13, 15, 26, 6), (17, 29, 16, 24))
    x0 = jnp.zeros_like(cnt) + ks0
    x1 = cnt + ks1
    for i in range(5):
        for r in rots[i % 2]:
            x0 = x0 + x1
            x1 = _rotl(x1, r)
            x1 = x0 ^ x1
        x0 = x0 + ks[(i + 1) % 3]
        x1 = x1 + ks[(i + 2) % 3] + jnp.int32(i + 1)
    return x0 ^ x1


def _body(blk_ref, xtm_ref, sig_ref, step_ref, x_hbm_ref, rev_ref, nxt_ref,
          xbuf_ref, copy_sem):
    j = pl.program_id(0)
    xtm = xtm_ref[...]
    any_masked = blk_ref[j] != 0
    dirty = j < 4
    for back in range(1, 5):
        dirty = jnp.logical_or(
            dirty, blk_ref[jnp.maximum(j - back, 0)] != 0)

    @pl.when(jnp.logical_and(jnp.logical_not(any_masked), dirty))
    def _():
        rev_ref[...] = jnp.zeros((_BR, _V), jnp.float32)

    @pl.when(jnp.logical_not(any_masked))
    def _():
        nxt_ref[...] = xtm_ref[...]

    @pl.when(any_masked)
    def _():
        for sub in range(_BR // _BS):
            r0 = sub * _BS
            copy = pltpu.make_async_copy(
                x_hbm_ref.at[pl.ds(j * _BR + r0, _BS), :], xbuf_ref, copy_sem)
            copy.start()
            copy.wait()
            x = xbuf_ref[...]
            score = jnp.exp(x)
            sig = sig_ref[pl.ds(r0, _BS), :]
            xtm_c = xtm_ref[pl.ds(r0, _BS), :]
            vv = jax.lax.broadcasted_iota(jnp.int32, (_BS, _V), 1)
            is_last = vv == _V - 1
            masked_row = xtm_c == _V - 1
            s = jnp.sum(jnp.where(is_last, 0.0, score), axis=1, keepdims=True)
            rev = sig * jnp.where(is_last, -s, score)
            rev = jnp.where(masked_row, rev, 0.0)
            rev_ref[pl.ds(r0, _BS), :] = rev
            row = (j * _BR + r0
                   + jax.lax.broadcasted_iota(jnp.int32, (_BS, _V), 0))
            bits = _threefry_bits(row * _V + vv)
            fbits = jax.lax.bitcast_convert_type(
                jax.lax.shift_right_logical(bits, 9) | jnp.int32(0x3F800000),
                jnp.float32) - 1.0
            u = jnp.maximum(fbits, 0.0)
            noise = 1e-6 - jnp.log(1e-6 + (1.0 - 1e-6) * u)
            step = step_ref[0]
            xt_prob = jnp.where(is_last, 1.0 + step * rev, step * rev)
            vals = xt_prob / noise
            m = jnp.max(vals, axis=1, keepdims=True)
            idx = jnp.min(jnp.where(vals == m, vv, _V), axis=1, keepdims=True)
            nxt_ref[pl.ds(r0, _BS), :] = jnp.where(
                masked_row, idx, xtm_c)


def kernel(output, xt, t, step_size):
    xt = xt.astype(jnp.int32)
    xtm = jnp.where(xt == _MASK_IDX, _V - 1, xt).reshape(_R, 1)
    blk = (jnp.max(xtm.reshape(_G, _BR), axis=1) == _V - 1).astype(jnp.int32)
    sigma = ((1.0 - _EPS) / (1.0 - (1.0 - _EPS) * t)).astype(jnp.float32)
    sigma_rows = jnp.repeat(sigma, _L).reshape(_R, 1)
    x2d = output.reshape(_R, _V)
    step = step_size.astype(jnp.float32)

    rev2d, nxt = pl.pallas_call(
        _body,
        grid=(_G,),
        in_specs=[
            pl.BlockSpec(memory_space=pltpu.SMEM),
            pl.BlockSpec((_BR, 1), lambda j: (j, 0)),
            pl.BlockSpec((_BR, 1), lambda j: (j, 0)),
            pl.BlockSpec(memory_space=pltpu.SMEM),
            pl.BlockSpec(memory_space=pl.ANY),
        ],
        out_specs=[
            pl.BlockSpec((_BR, _V), lambda j: (j, 0)),
            pl.BlockSpec((_BR, 1), lambda j: (j, 0)),
        ],
        out_shape=[
            jax.ShapeDtypeStruct((_R, _V), jnp.float32),
            jax.ShapeDtypeStruct((_R, 1), jnp.int32),
        ],
        scratch_shapes=[
            pltpu.VMEM((_BS, _V), jnp.float32),
            pltpu.SemaphoreType.DMA,
        ],
        compiler_params=pltpu.CompilerParams(
            vmem_limit_bytes=120 * 1024 * 1024),
    )(blk, xtm, sigma_rows, step, x2d)

    new_xt = jnp.where(nxt.reshape(_B, _L) == _V - 1, _MASK_IDX,
                       nxt.reshape(_B, _L))
    return new_xt, rev2d.reshape(_B, _L, _V)


# speedup vs baseline: 6.3210x; 1.0275x over previous
import jax
import jax.numpy as jnp
from jax.experimental import pallas as pl
from jax.experimental.pallas import tpu as pltpu

_V = 100001
_MASK_IDX = -1
_EPS = 1e-3
_B = 16
_L = 16
_R = _B * _L
_BR = 16
_BS = 16
_G = _R // _BR


def _rotl(x, d):
    return jax.lax.shift_left(x, d) | jax.lax.shift_right_logical(x, 32 - d)


def _threefry_bits(cnt):
    ks0 = jnp.int32(0)
    ks1 = jnp.int32(1)
    ks2 = jnp.int32(0x1BD11BDB)
    ks = (ks0, ks1, ks2)
    rots = ((13, 15, 26, 6), (17, 29, 16, 24))
    x0 = jnp.zeros_like(cnt) + ks0
    x1 = cnt + ks1
    for i in range(5):
        for r in rots[i % 2]:
            x0 = x0 + x1
            x1 = _rotl(x1, r)
            x1 = x0 ^ x1
        x0 = x0 + ks[(i + 1) % 3]
        x1 = x1 + ks[(i + 2) % 3] + jnp.int32(i + 1)
    return x0 ^ x1


def _body(blk_ref, xtm_ref, sig_ref, step_ref, x_hbm_ref, rev_ref, nxt_ref,
          xbuf_ref, copy_sem):
    j = pl.program_id(0)
    xtm = xtm_ref[...]
    any_masked = blk_ref[j] != 0
    dirty = j < 4
    for back in range(1, 5):
        dirty = jnp.logical_or(
            dirty, blk_ref[jnp.maximum(j - back, 0)] != 0)

    @pl.when(jnp.logical_and(jnp.logical_not(any_masked), dirty))
    def _():
        rev_ref[...] = jnp.zeros((_BR, _V), jnp.float32)

    @pl.when(jnp.logical_not(any_masked))
    def _():
        nxt_ref[...] = xtm_ref[...]

    @pl.when(any_masked)
    def _():
        for sub in range(_BR // _BS):
            r0 = sub * _BS
            copy = pltpu.make_async_copy(
                x_hbm_ref.at[pl.ds(j * _BR + r0, _BS), :], xbuf_ref, copy_sem)
            copy.start()
            copy.wait()
            x = xbuf_ref[...]
            score = jnp.exp(x)
            sig = sig_ref[pl.ds(r0, _BS), :]
            xtm_c = xtm_ref[pl.ds(r0, _BS), :]
            vv = jax.lax.broadcasted_iota(jnp.int32, (_BS, _V), 1)
            is_last = vv == _V - 1
            masked_row = xtm_c == _V - 1
            s = jnp.sum(jnp.where(is_last, 0.0, score), axis=1, keepdims=True)
            rev = sig * jnp.where(is_last, -s, score)
            rev = jnp.where(masked_row, rev, 0.0)
            rev_ref[pl.ds(r0, _BS), :] = rev
            row = (j * _BR + r0
                   + jax.lax.broadcasted_iota(jnp.int32, (_BS, _V), 0))
            bits = _threefry_bits(row * _V + vv)
            fbits = jax.lax.bitcast_convert_type(
                jax.lax.shift_right_logical(bits, 9) | jnp.int32(0x3F800000),
                jnp.float32) - 1.0
            u = jnp.maximum(fbits, 0.0)
            noise = 1e-6 - jnp.log(1e-6 + (1.0 - 1e-6) * u)
            step = step_ref[0]
            xt_prob = jnp.where(is_last, 1.0 + step * rev, step * rev)
            vals = xt_prob / noise
            m = jnp.max(vals, axis=1, keepdims=True)
            idx = jnp.min(jnp.where(vals == m, vv, _V), axis=1, keepdims=True)
            nxt_ref[pl.ds(r0, _BS), :] = jnp.where(
                masked_row, idx, xtm_c)


def kernel(output, xt, t, step_size):
    xt = xt.astype(jnp.int32)
    xtm = jnp.where(xt == _MASK_IDX, _V - 1, xt).reshape(_R, 1)
    blk = (jnp.max(xtm.reshape(_G, _BR), axis=1) == _V - 1).astype(jnp.int32)
    sigma = ((1.0 - _EPS) / (1.0 - (1.0 - _EPS) * t)).astype(jnp.float32)
    sigma_rows = jnp.repeat(sigma, _L).reshape(_R, 1)
    x2d = output.reshape(_R, _V)
    step = step_size.astype(jnp.float32)

    rev2d, nxt = pl.pallas_call(
        _body,
        grid=(_G,),
        in_specs=[
            pl.BlockSpec(memory_space=pltpu.SMEM),
            pl.BlockSpec((_BR, 1), lambda j: (j, 0)),
            pl.BlockSpec((_BR, 1), lambda j: (j, 0)),
            pl.BlockSpec(memory_space=pltpu.SMEM),
            pl.BlockSpec(memory_space=pl.ANY),
        ],
        out_specs=[
            pl.BlockSpec((_BR, _V), lambda j: (j, 0)),
            pl.BlockSpec((_BR, 1), lambda j: (j, 0)),
        ],
        out_shape=[
            jax.ShapeDtypeStruct((_R, _V), jnp.float32),
            jax.ShapeDtypeStruct((_R, 1), jnp.int32),
        ],
        scratch_shapes=[
            pltpu.VMEM((_BS, _V), jnp.float32),
            pltpu.SemaphoreType.DMA,
        ],
        compiler_params=pltpu.CompilerParams(
            vmem_limit_bytes=120 * 1024 * 1024),
    )(blk, xtm, sigma_rows, step, x2d)

    new_xt = jnp.where(nxt.reshape(_B, _L) == _V - 1, _MASK_IDX,
                       nxt.reshape(_B, _L))
    return new_xt, rev2d.reshape(_B, _L, _V)


# speedup vs baseline: 6.7835x; 1.0732x over previous
import jax
import jax.numpy as jnp
from jax.experimental import pallas as pl
from jax.experimental.pallas import tpu as pltpu

_V = 100001
_MASK_IDX = -1
_EPS = 1e-3
_B = 16
_L = 16
_R = _B * _L
_BR = 8
_BS = 8
_G = _R // _BR


def _rotl(x, d):
    return jax.lax.shift_left(x, d) | jax.lax.shift_right_logical(x, 32 - d)


def _threefry_bits(cnt):
    ks0 = jnp.int32(0)
    ks1 = jnp.int32(1)
    ks2 = jnp.int32(0x1BD11BDB)
    ks = (ks0, ks1, ks2)
    rots = ((13, 15, 26, 6), (17, 29, 16, 24))
    x0 = jnp.zeros_like(cnt) + ks0
    x1 = cnt + ks1
    for i in range(5):
        for r in rots[i % 2]:
            x0 = x0 + x1
            x1 = _rotl(x1, r)
            x1 = x0 ^ x1
        x0 = x0 + ks[(i + 1) % 3]
        x1 = x1 + ks[(i + 2) % 3] + jnp.int32(i + 1)
    return x0 ^ x1


def _body(blk_ref, xtm_ref, sig_ref, step_ref, x_hbm_ref, rev_ref, nxt_ref,
          xbuf_ref, copy_sem):
    j = pl.program_id(0)
    xtm = xtm_ref[...]
    any_masked = blk_ref[j] != 0
    dirty = j < 4
    for back in range(1, 5):
        dirty = jnp.logical_or(
            dirty, blk_ref[jnp.maximum(j - back, 0)] != 0)

    @pl.when(jnp.logical_and(jnp.logical_not(any_masked), dirty))
    def _():
        rev_ref[...] = jnp.zeros((_BR, _V), jnp.float32)

    @pl.when(jnp.logical_not(any_masked))
    def _():
        nxt_ref[...] = xtm_ref[...]

    @pl.when(any_masked)
    def _():
        for sub in range(_BR // _BS):
            r0 = sub * _BS
            copy = pltpu.make_async_copy(
                x_hbm_ref.at[pl.ds(j * _BR + r0, _BS), :], xbuf_ref, copy_sem)
            copy.start()
            copy.wait()
            x = xbuf_ref[...]
            score = jnp.exp(x)
            sig = sig_ref[pl.ds(r0, _BS), :]
            xtm_c = xtm_ref[pl.ds(r0, _BS), :]
            vv = jax.lax.broadcasted_iota(jnp.int32, (_BS, _V), 1)
            is_last = vv == _V - 1
            masked_row = xtm_c == _V - 1
            s = jnp.sum(jnp.where(is_last, 0.0, score), axis=1, keepdims=True)
            rev = sig * jnp.where(is_last, -s, score)
            rev = jnp.where(masked_row, rev, 0.0)
            rev_ref[pl.ds(r0, _BS), :] = rev
            row = (j * _BR + r0
                   + jax.lax.broadcasted_iota(jnp.int32, (_BS, _V), 0))
            bits = _threefry_bits(row * _V + vv)
            fbits = jax.lax.bitcast_convert_type(
                jax.lax.shift_right_logical(bits, 9) | jnp.int32(0x3F800000),
                jnp.float32) - 1.0
            u = jnp.maximum(fbits, 0.0)
            noise = 1e-6 - jnp.log(1e-6 + (1.0 - 1e-6) * u)
            step = step_ref[0]
            xt_prob = jnp.where(is_last, 1.0 + step * rev, step * rev)
            vals = xt_prob / noise
            m = jnp.max(vals, axis=1, keepdims=True)
            idx = jnp.min(jnp.where(vals == m, vv, _V), axis=1, keepdims=True)
            nxt_ref[pl.ds(r0, _BS), :] = jnp.where(
                masked_row, idx, xtm_c)


def kernel(output, xt, t, step_size):
    xt = xt.astype(jnp.int32)
    xtm = jnp.where(xt == _MASK_IDX, _V - 1, xt).reshape(_R, 1)
    blk = (jnp.max(xtm.reshape(_G, _BR), axis=1) == _V - 1).astype(jnp.int32)
    sigma = ((1.0 - _EPS) / (1.0 - (1.0 - _EPS) * t)).astype(jnp.float32)
    sigma_rows = jnp.repeat(sigma, _L).reshape(_R, 1)
    x2d = output.reshape(_R, _V)
    step = step_size.astype(jnp.float32)

    rev2d, nxt = pl.pallas_call(
        _body,
        grid=(_G,),
        in_specs=[
            pl.BlockSpec(memory_space=pltpu.SMEM),
            pl.BlockSpec((_BR, 1), lambda j: (j, 0)),
            pl.BlockSpec((_BR, 1), lambda j: (j, 0)),
            pl.BlockSpec(memory_space=pltpu.SMEM),
            pl.BlockSpec(memory_space=pl.ANY),
        ],
        out_specs=[
            pl.BlockSpec((_BR, _V), lambda j: (j, 0)),
            pl.BlockSpec((_BR, 1), lambda j: (j, 0)),
        ],
        out_shape=[
            jax.ShapeDtypeStruct((_R, _V), jnp.float32),
            jax.ShapeDtypeStruct((_R, 1), jnp.int32),
        ],
        scratch_shapes=[
            pltpu.VMEM((_BS, _V), jnp.float32),
            pltpu.SemaphoreType.DMA,
        ],
        compiler_params=pltpu.CompilerParams(
            vmem_limit_bytes=120 * 1024 * 1024),
    )(blk, xtm, sigma_rows, step, x2d)

    new_xt = jnp.where(nxt.reshape(_B, _L) == _V - 1, _MASK_IDX,
                       nxt.reshape(_B, _L))
    return new_xt, rev2d.reshape(_B, _L, _V)
